# Initial kernel scaffold; baseline (speedup 1.0000x reference)
import jax
import jax.numpy as jnp
from jax.experimental import pallas as pl


def kernel(hidden_states, gate_weight, e_score_correction_bias, w1, w2, w3, ws1, ws2, ws3):
    raise NotImplementedError("write your pallas kernel here")


# speedup vs baseline: 1.0787x; 1.0787x over previous
import functools

import jax
import jax.numpy as jnp
from jax.experimental import pallas as pl
from jax.experimental.pallas import tpu as pltpu

E = 8
TOP_K = 2
N_GROUP = 4
TOPK_GROUP = 2
SCALE = 2.5

R = 256
RS = 512


def _route(x_flat, gate_weight, bias):
    router_logits = x_flat.astype(jnp.float32) @ gate_weight.T
    scores = jax.nn.sigmoid(router_logits)
    scores_for_choice = scores + bias[None, :]
    t = scores_for_choice.shape[0]
    grp = scores_for_choice.reshape(t, N_GROUP, E // N_GROUP)
    top2_vals, _ = jax.lax.top_k(grp, 2)
    group_scores = top2_vals.sum(axis=-1)
    _, group_idx = jax.lax.top_k(group_scores, TOPK_GROUP)
    group_mask = jnp.zeros_like(group_scores).at[
        jnp.arange(t)[:, None], group_idx
    ].set(1.0)
    score_mask = jnp.repeat(group_mask, E // N_GROUP, axis=1)
    masked_scores = jnp.where(score_mask > 0, scores_for_choice, 0.0)
    _, topk_idx = jax.lax.top_k(masked_scores, TOP_K)
    topk_w = jnp.take_along_axis(scores, topk_idx, axis=1)
    topk_w = topk_w / (topk_w.sum(axis=-1, keepdims=True) + 1e-20)
    topk_w = topk_w * SCALE
    return topk_idx, topk_w


def _gmm_body(te_ref, xg_ref, w1_ref, w3_ref, w2_ref, coef_ref, out_ref):
    xr = xg_ref[...]
    w1 = w1_ref[0].astype(jnp.bfloat16)
    w3 = w3_ref[0].astype(jnp.bfloat16)
    w2 = w2_ref[0].astype(jnp.bfloat16)
    nt = (((1,), (1,)), ((), ()))
    a = jax.lax.dot_general(xr, w1, nt, preferred_element_type=jnp.float32)
    b = jax.lax.dot_general(xr, w3, nt, preferred_element_type=jnp.float32)
    h = (a * jax.nn.sigmoid(a)) * b
    h = (h * coef_ref[...]).astype(jnp.bfloat16)
    y = jax.lax.dot_general(h, w2, nt, preferred_element_type=jnp.float32)
    out_ref[...] = y.astype(jnp.bfloat16)


def _shared_body(x_ref, ws1_ref, ws3_ref, ws2_ref, ysum_ref, out_ref):
    xr = x_ref[...]
    w1 = ws1_ref[...].astype(jnp.bfloat16)
    w3 = ws3_ref[...].astype(jnp.bfloat16)
    w2 = ws2_ref[...].astype(jnp.bfloat16)
    nt = (((1,), (1,)), ((), ()))
    a = jax.lax.dot_general(xr, w1, nt, preferred_element_type=jnp.float32)
    b = jax.lax.dot_general(xr, w3, nt, preferred_element_type=jnp.float32)
    h = ((a * jax.nn.sigmoid(a)) * b).astype(jnp.bfloat16)
    y = jax.lax.dot_general(h, w2, nt, preferred_element_type=jnp.float32)
    out_ref[...] = y + ysum_ref[...]


def kernel(hidden_states, gate_weight, e_score_correction_bias,
           w1, w2, w3, ws1, ws2, ws3):
    b, s, hdim = hidden_states.shape
    t = b * s
    d_ff = w1.shape[1]
    d_sh = ws1.shape[0]
    x = hidden_states.reshape(t, hdim)
    x16 = x.astype(jnp.bfloat16)

    topk_idx, topk_w = _route(x, gate_weight, e_score_correction_bias)

    p_max = t * TOP_K + E * R
    n_tiles = p_max // R
    onehot = jax.nn.one_hot(topk_idx, E, dtype=jnp.int32).sum(axis=1)
    rank = jnp.cumsum(onehot, axis=0) - onehot
    counts = jnp.sum(onehot, axis=0)
    padded = ((counts + R - 1) // R) * R
    pad_off = jnp.concatenate(
        [jnp.zeros((1,), jnp.int32), jnp.cumsum(padded)[:-1].astype(jnp.int32)])
    slots = jnp.take_along_axis(pad_off[None, :] + rank, topk_idx, axis=1)
    tile_expert = (jnp.sum(
        pad_off[None, :] <= (jnp.arange(n_tiles, dtype=jnp.int32) * R)[:, None],
        axis=1) - 1).astype(jnp.int32)

    sf = slots.reshape(-1)
    tok = jnp.broadcast_to(
        jnp.arange(t, dtype=jnp.int32)[:, None], (t, TOP_K)).reshape(-1)
    src_token = jnp.zeros((p_max,), jnp.int32).at[sf].set(tok)
    pair_coef = jnp.zeros((p_max, 1), jnp.float32).at[sf, 0].set(topk_w.reshape(-1))

    xg = jnp.take(x16, src_token, axis=0)

    yg = pl.pallas_call(
        _gmm_body,
        grid_spec=pltpu.PrefetchScalarGridSpec(
            num_scalar_prefetch=1,
            grid=(n_tiles,),
            in_specs=[
                pl.BlockSpec((R, hdim), lambda i, te: (i, 0)),
                pl.BlockSpec((1, d_ff, hdim), lambda i, te: (te[i], 0, 0)),
                pl.BlockSpec((1, d_ff, hdim), lambda i, te: (te[i], 0, 0)),
                pl.BlockSpec((1, hdim, d_ff), lambda i, te: (te[i], 0, 0)),
                pl.BlockSpec((R, 1), lambda i, te: (i, 0)),
            ],
            out_specs=pl.BlockSpec((R, hdim), lambda i, te: (i, 0)),
        ),
        out_shape=jax.ShapeDtypeStruct((p_max, hdim), jnp.bfloat16),
    )(tile_expert, xg, w1, w3, w2, pair_coef)

    ysum = (jnp.take(yg, slots[:, 0], axis=0).astype(jnp.float32)
            + jnp.take(yg, slots[:, 1], axis=0).astype(jnp.float32))

    out = pl.pallas_call(
        _shared_body,
        grid=(t // RS,),
        in_specs=[
            pl.BlockSpec((RS, hdim), lambda i: (i, 0)),
            pl.BlockSpec((d_sh, hdim), lambda i: (0, 0)),
            pl.BlockSpec((d_sh, hdim), lambda i: (0, 0)),
            pl.BlockSpec((hdim, d_sh), lambda i: (0, 0)),
            pl.BlockSpec((RS, hdim), lambda i: (i, 0)),
        ],
        out_specs=pl.BlockSpec((RS, hdim), lambda i: (i, 0)),
        out_shape=jax.ShapeDtypeStruct((t, hdim), jnp.float32),
    )(x16, ws1, ws3, ws2, ysum)

    return out.reshape(b, s, hdim).astype(hidden_states.dtype)


# speedup vs baseline: 1.2804x; 1.1870x over previous
import functools

import jax
import jax.numpy as jnp
from jax import lax
from jax.experimental import pallas as pl
from jax.experimental.pallas import tpu as pltpu
from jax.experimental.pallas import tpu_sc as plsc

E = 8
TOP_K = 2
N_GROUP = 4
TOPK_GROUP = 2
SCALE = 2.5

R = 256
RS = 256
NC = 2
NS = 16
NW = NC * NS


def _route(x_flat, gate_weight, bias):
    router_logits = x_flat.astype(jnp.float32) @ gate_weight.T
    scores = jax.nn.sigmoid(router_logits)
    scores_for_choice = scores + bias[None, :]
    t = scores_for_choice.shape[0]
    grp = scores_for_choice.reshape(t, N_GROUP, E // N_GROUP)
    top2_vals, _ = jax.lax.top_k(grp, 2)
    group_scores = top2_vals.sum(axis=-1)
    _, group_idx = jax.lax.top_k(group_scores, TOPK_GROUP)
    group_mask = jnp.zeros_like(group_scores).at[
        jnp.arange(t)[:, None], group_idx
    ].set(1.0)
    score_mask = jnp.repeat(group_mask, E // N_GROUP, axis=1)
    masked_scores = jnp.where(score_mask > 0, scores_for_choice, 0.0)
    _, topk_idx = jax.lax.top_k(masked_scores, TOP_K)
    topk_w = jnp.take_along_axis(scores, topk_idx, axis=1)
    topk_w = topk_w / (topk_w.sum(axis=-1, keepdims=True) + 1e-20)
    topk_w = topk_w * SCALE
    return topk_idx, topk_w


def _dispatch_sc(x3, slots_t, p_max):
    t = x3.shape[0]
    per_w = t // NW
    ch = 32
    n_ch = per_w // ch
    mesh = plsc.VectorSubcoreMesh(core_axis_name="c", subcore_axis_name="s")

    @functools.partial(
        pl.kernel, mesh=mesh,
        out_type=jax.ShapeDtypeStruct((p_max, 16, 128), jnp.float32),
        scratch_types=[
            pltpu.VMEM((ch,), jnp.int32),
            pltpu.VMEM((ch,), jnp.int32),
            pltpu.VMEM((ch, 16, 128), jnp.float32),
            pltpu.SemaphoreType.DMA,
            pltpu.SemaphoreType.DMA,
        ],
    )
    def run(x_hbm, st_hbm, xg_hbm, idx0, idx1, rows, sem0, sem1):
        wid = lax.axis_index("s") * NC + lax.axis_index("c")
        for c in range(n_ch):
            base = wid * per_w + c * ch
            pltpu.sync_copy(st_hbm.at[0, pl.ds(base, ch)], idx0)
            pltpu.sync_copy(st_hbm.at[1, pl.ds(base, ch)], idx1)
            pltpu.sync_copy(x_hbm.at[pl.ds(base, ch)], rows)
            c0 = pltpu.async_copy(rows, xg_hbm.at[idx0], sem0)
            c1 = pltpu.async_copy(rows, xg_hbm.at[idx1], sem1)
            c0.wait()
            c1.wait()

    return run(x3, slots_t)


def _combine_sc(yg3, slots_t):
    t = slots_t.shape[1]
    per_w = t // NW
    ch = 16
    n_ch = per_w // ch
    mesh = plsc.VectorSubcoreMesh(core_axis_name="c", subcore_axis_name="s")
    oshape = jax.ShapeDtypeStruct((t, 16, 128), jnp.float32)

    @functools.partial(
        pl.kernel, mesh=mesh,
        out_type=(oshape, oshape),
        scratch_types=[
            pltpu.VMEM((ch,), jnp.int32),
            pltpu.VMEM((ch,), jnp.int32),
            pltpu.VMEM((ch, 16, 128), jnp.float32),
            pltpu.VMEM((ch, 16, 128), jnp.float32),
            pltpu.SemaphoreType.DMA,
            pltpu.SemaphoreType.DMA,
        ],
    )
    def run(yg_hbm, st_hbm, y0_hbm, y1_hbm, idx0, idx1, rows0, rows1,
            sem0, sem1):
        wid = lax.axis_index("s") * NC + lax.axis_index("c")
        for c in range(n_ch):
            base = wid * per_w + c * ch
            pltpu.sync_copy(st_hbm.at[0, pl.ds(base, ch)], idx0)
            pltpu.sync_copy(st_hbm.at[1, pl.ds(base, ch)], idx1)
            g0 = pltpu.async_copy(yg_hbm.at[idx0], rows0, sem0)
            g1 = pltpu.async_copy(yg_hbm.at[idx1], rows1, sem1)
            g0.wait()
            pltpu.sync_copy(rows0, y0_hbm.at[pl.ds(base, ch)])
            g1.wait()
            pltpu.sync_copy(rows1, y1_hbm.at[pl.ds(base, ch)])

    return run(yg3, slots_t)


def _gmm_body(te_ref, xg_ref, w1_ref, w3_ref, w2_ref, out_ref):
    xr = xg_ref[...].astype(jnp.bfloat16)
    w1 = w1_ref[0].astype(jnp.bfloat16)
    w3 = w3_ref[0].astype(jnp.bfloat16)
    w2 = w2_ref[0].astype(jnp.bfloat16)
    nt = (((1,), (1,)), ((), ()))
    a = jax.lax.dot_general(xr, w1, nt, preferred_element_type=jnp.float32)
    b = jax.lax.dot_general(xr, w3, nt, preferred_element_type=jnp.float32)
    h = ((a * jax.nn.sigmoid(a)) * b).astype(jnp.bfloat16)
    y = jax.lax.dot_general(h, w2, nt, preferred_element_type=jnp.float32)
    out_ref[...] = y


def _shared_body(x_ref, ws1_ref, ws3_ref, ws2_ref, y0_ref, y1_ref,
                 w0_ref, w1c_ref, out_ref):
    xr = x_ref[...].astype(jnp.bfloat16)
    w1 = ws1_ref[...].astype(jnp.bfloat16)
    w3 = ws3_ref[...].astype(jnp.bfloat16)
    w2 = ws2_ref[...].astype(jnp.bfloat16)
    nt = (((1,), (1,)), ((), ()))
    a = jax.lax.dot_general(xr, w1, nt, preferred_element_type=jnp.float32)
    b = jax.lax.dot_general(xr, w3, nt, preferred_element_type=jnp.float32)
    h = ((a * jax.nn.sigmoid(a)) * b).astype(jnp.bfloat16)
    y = jax.lax.dot_general(h, w2, nt, preferred_element_type=jnp.float32)
    y = y + w0_ref[...] * y0_ref[...]
    y = y + w1c_ref[...] * y1_ref[...]
    out_ref[...] = y


def kernel(hidden_states, gate_weight, e_score_correction_bias,
           w1, w2, w3, ws1, ws2, ws3):
    b, s, hdim = hidden_states.shape
    t = b * s
    d_ff = w1.shape[1]
    d_sh = ws1.shape[0]
    x = hidden_states.reshape(t, hdim)

    topk_idx, topk_w = _route(x, gate_weight, e_score_correction_bias)

    p_max = t * TOP_K + E * R
    n_tiles = p_max // R
    onehot = jax.nn.one_hot(topk_idx, E, dtype=jnp.int32).sum(axis=1)
    rank = jnp.cumsum(onehot, axis=0) - onehot
    counts = jnp.sum(onehot, axis=0)
    padded = ((counts + R - 1) // R) * R
    pad_off = jnp.concatenate(
        [jnp.zeros((1,), jnp.int32), jnp.cumsum(padded)[:-1].astype(jnp.int32)])
    slots = jnp.take_along_axis(pad_off[None, :] + rank, topk_idx, axis=1)
    slots_t = slots.T
    tile_expert = (jnp.sum(
        pad_off[None, :] <= (jnp.arange(n_tiles, dtype=jnp.int32) * R)[:, None],
        axis=1) - 1).astype(jnp.int32)

    xg = _dispatch_sc(x.reshape(t, 16, 128), slots_t, p_max)

    yg = pl.pallas_call(
        _gmm_body,
        grid_spec=pltpu.PrefetchScalarGridSpec(
            num_scalar_prefetch=1,
            grid=(n_tiles,),
            in_specs=[
                pl.BlockSpec((R, hdim), lambda i, te: (i, 0)),
                pl.BlockSpec((1, d_ff, hdim), lambda i, te: (te[i], 0, 0)),
                pl.BlockSpec((1, d_ff, hdim), lambda i, te: (te[i], 0, 0)),
                pl.BlockSpec((1, hdim, d_ff), lambda i, te: (te[i], 0, 0)),
            ],
            out_specs=pl.BlockSpec((R, hdim), lambda i, te: (i, 0)),
        ),
        out_shape=jax.ShapeDtypeStruct((p_max, hdim), jnp.float32),
    )(tile_expert, xg.reshape(p_max, hdim), w1, w3, w2)

    y0, y1 = _combine_sc(yg.reshape(p_max, 16, 128), slots_t)

    out = pl.pallas_call(
        _shared_body,
        grid=(t // RS,),
        in_specs=[
            pl.BlockSpec((RS, hdim), lambda i: (i, 0)),
            pl.BlockSpec((d_sh, hdim), lambda i: (0, 0)),
            pl.BlockSpec((d_sh, hdim), lambda i: (0, 0)),
            pl.BlockSpec((hdim, d_sh), lambda i: (0, 0)),
            pl.BlockSpec((RS, hdim), lambda i: (i, 0)),
            pl.BlockSpec((RS, hdim), lambda i: (i, 0)),
            pl.BlockSpec((RS, 1), lambda i: (i, 0)),
            pl.BlockSpec((RS, 1), lambda i: (i, 0)),
        ],
        out_specs=pl.BlockSpec((RS, hdim), lambda i: (i, 0)),
        out_shape=jax.ShapeDtypeStruct((t, hdim), jnp.float32),
    )(x, ws1, ws3, ws2, y0.reshape(t, hdim), y1.reshape(t, hdim),
      topk_w[:, 0:1], topk_w[:, 1:2])

    return out.reshape(b, s, hdim).astype(hidden_states.dtype)
